# baseline (device time: 22335 ns/iter reference)
import jax
import jax.numpy as jnp
from jax import lax
from jax.experimental import pallas as pl
from jax.experimental.pallas import tpu as pltpu

N_DEV = 4
B, SQ, SKV, HQ_LOCAL, DH = 2, 128, 128, 4, 64
D_MODEL = 512
D_HEADS = HQ_LOCAL * DH
M = B * SQ


def kernel(x, Wq, K_ext, V_ext, Wo):
    my = lax.axis_index("i")
    Wq_s = lax.dynamic_slice_in_dim(Wq, my * D_HEADS, D_HEADS, axis=1)
    Wo_s = lax.dynamic_slice_in_dim(Wo, my * D_HEADS, D_HEADS, axis=0)
    x2d = x.reshape(M, D_MODEL)

    def body(x_ref, wq_ref, k_ref, v_ref, wo_ref, out_ref,
             comm_ref, send_sems, recv_sems):
        my_pos = lax.axis_index("i")
        left = (my_pos - 1) % N_DEV
        right = (my_pos + 1) % N_DEV

        barrier_sem = pltpu.get_barrier_semaphore()
        for nbr in [left, right]:
            pl.semaphore_signal(
                barrier_sem, inc=1,
                device_id=(nbr,), device_id_type=pl.DeviceIdType.MESH,
            )
        pl.semaphore_wait(barrier_sem, 2)

        f32 = jnp.float32
        xb = x_ref[:].astype(jnp.bfloat16)
        wqb = wq_ref[:].astype(jnp.bfloat16)
        q_all = lax.dot(xb, wqb, preferred_element_type=f32)
        q_all = q_all.astype(jnp.bfloat16)

        rows = lax.broadcasted_iota(jnp.int32, (SQ, SKV), 0)
        cols = lax.broadcasted_iota(jnp.int32, (SQ, SKV), 1)
        mask = (cols // 64) <= (rows // 64)

        ctx_rows = []
        for b in range(B):
            ctx_heads = []
            for h in range(HQ_LOCAL):
                q = q_all[b * SQ:(b + 1) * SQ, h * DH:(h + 1) * DH]
                k = k_ref[b, :, h, :].astype(jnp.bfloat16)
                v = v_ref[b, :, h, :].astype(jnp.bfloat16)
                s = lax.dot_general(
                    q, k, (((1,), (1,)), ((), ())),
                    preferred_element_type=f32,
                ) * 0.125
                s = jnp.where(mask, s, -1e9)
                s = s - jnp.max(s, axis=-1, keepdims=True)
                w = jnp.exp(s)
                w = w / jnp.sum(w, axis=-1, keepdims=True)
                ctx_heads.append(
                    lax.dot(w.astype(jnp.bfloat16), v,
                            preferred_element_type=f32)
                )
            ctx_rows.append(jnp.concatenate(ctx_heads, axis=1))
        ctx = jnp.concatenate(ctx_rows, axis=0).astype(jnp.bfloat16)

        wob = wo_ref[:].astype(jnp.bfloat16)
        partial = lax.dot(ctx, wob, preferred_element_type=f32)

        comm_ref[0, :, :] = partial.astype(jnp.bfloat16)
        acc = partial
        for h in range(N_DEV - 1):
            rdma = pltpu.make_async_remote_copy(
                src_ref=comm_ref.at[h],
                dst_ref=comm_ref.at[h + 1],
                send_sem=send_sems.at[h],
                recv_sem=recv_sems.at[h],
                device_id=(right,),
                device_id_type=pl.DeviceIdType.MESH,
            )
            rdma.start()
            rdma.wait()
            acc = acc + comm_ref[h + 1, :, :].astype(f32)
        out_ref[:] = acc

    out2d = pl.pallas_call(
        body,
        out_shape=jax.ShapeDtypeStruct((M, D_MODEL), jnp.float32),
        in_specs=[pl.BlockSpec(memory_space=pltpu.VMEM)] * 5,
        out_specs=pl.BlockSpec(memory_space=pltpu.VMEM),
        scratch_shapes=[
            pltpu.VMEM((N_DEV, M, D_MODEL), jnp.bfloat16),
            pltpu.SemaphoreType.DMA((N_DEV - 1,)),
            pltpu.SemaphoreType.DMA((N_DEV - 1,)),
        ],
        compiler_params=pltpu.CompilerParams(collective_id=0),
    )(x2d, Wq_s, K_ext, V_ext, Wo_s)
    return out2d.reshape(B, SQ, D_MODEL)


# device time: 16758 ns/iter; 1.3328x vs baseline; 1.3328x over previous
import jax
import jax.numpy as jnp
from jax import lax
from jax.experimental import pallas as pl
from jax.experimental.pallas import tpu as pltpu

N_DEV = 4
B, SQ, SKV, HQ_LOCAL, DH = 2, 128, 128, 4, 64
D_MODEL = 512
D_HEADS = HQ_LOCAL * DH
M = B * SQ


def kernel(x, Wq, K_ext, V_ext, Wo):
    my = lax.axis_index("i")
    Wq_s = lax.dynamic_slice_in_dim(Wq, my * D_HEADS, D_HEADS, axis=1)
    Wo_s = lax.dynamic_slice_in_dim(Wo, my * D_HEADS, D_HEADS, axis=0)
    x2d = x.reshape(M, D_MODEL)

    def body(x_ref, wq_ref, k_ref, v_ref, wo_ref, out_ref,
             buf_a, buf_b, comm_ref, send_sems, recv_sems):
        my_pos = lax.axis_index("i")
        p1 = my_pos ^ 1
        p2 = 3 - my_pos

        barrier_sem = pltpu.get_barrier_semaphore()
        for nbr in [p1, p2]:
            pl.semaphore_signal(
                barrier_sem, inc=1,
                device_id=(nbr,), device_id_type=pl.DeviceIdType.MESH,
            )

        f32 = jnp.float32
        xb = x_ref[:].astype(jnp.bfloat16)
        wqb = wq_ref[:].astype(jnp.bfloat16)
        q_all = lax.dot(xb, wqb, preferred_element_type=f32)
        q_all = q_all.astype(jnp.bfloat16)

        rows = lax.broadcasted_iota(jnp.int32, (SQ, SKV), 0)
        cols = lax.broadcasted_iota(jnp.int32, (SQ, SKV), 1)
        mask = (cols // 64) <= (rows // 64)

        ctx_rows = []
        for b in range(B):
            ctx_heads = []
            for h in range(HQ_LOCAL):
                q = q_all[b * SQ:(b + 1) * SQ, h * DH:(h + 1) * DH]
                k = k_ref[b, :, h, :].astype(jnp.bfloat16)
                v = v_ref[b, :, h, :].astype(jnp.bfloat16)
                s = lax.dot_general(
                    q, k, (((1,), (1,)), ((), ())),
                    preferred_element_type=f32,
                ) * 0.125
                w = jnp.exp(jnp.where(mask, s, -1e9))
                w = w / jnp.sum(w, axis=-1, keepdims=True)
                ctx_heads.append(
                    lax.dot(w.astype(jnp.bfloat16), v,
                            preferred_element_type=f32)
                )
            ctx_rows.append(jnp.concatenate(ctx_heads, axis=1))
        ctx = jnp.concatenate(ctx_rows, axis=0).astype(jnp.bfloat16)

        wob = wo_ref[:].astype(jnp.bfloat16)
        partial = lax.dot(ctx, wob, preferred_element_type=f32)
        buf_a[:, :] = partial.astype(jnp.bfloat16)

        pl.semaphore_wait(barrier_sem, 2)

        rdma1 = pltpu.make_async_remote_copy(
            src_ref=buf_a, dst_ref=comm_ref.at[0],
            send_sem=send_sems.at[0], recv_sem=recv_sems.at[0],
            device_id=(p1,), device_id_type=pl.DeviceIdType.MESH,
        )
        rdma1.start()
        rdma1.wait_recv()
        sum1 = partial + comm_ref[0, :, :].astype(f32)
        buf_b[:, :] = sum1.astype(jnp.bfloat16)

        rdma2 = pltpu.make_async_remote_copy(
            src_ref=buf_b, dst_ref=comm_ref.at[1],
            send_sem=send_sems.at[1], recv_sem=recv_sems.at[1],
            device_id=(p2,), device_id_type=pl.DeviceIdType.MESH,
        )
        rdma2.start()
        rdma2.wait_recv()
        out_ref[:] = sum1 + comm_ref[1, :, :].astype(f32)

        rdma1.wait_send()
        rdma2.wait_send()

    out2d = pl.pallas_call(
        body,
        out_shape=jax.ShapeDtypeStruct((M, D_MODEL), jnp.float32),
        in_specs=[pl.BlockSpec(memory_space=pltpu.VMEM)] * 5,
        out_specs=pl.BlockSpec(memory_space=pltpu.VMEM),
        scratch_shapes=[
            pltpu.VMEM((M, D_MODEL), jnp.bfloat16),
            pltpu.VMEM((M, D_MODEL), jnp.bfloat16),
            pltpu.VMEM((2, M, D_MODEL), jnp.bfloat16),
            pltpu.SemaphoreType.DMA((2,)),
            pltpu.SemaphoreType.DMA((2,)),
        ],
        compiler_params=pltpu.CompilerParams(collective_id=0),
    )(x2d, Wq_s, K_ext, V_ext, Wo_s)
    return out2d.reshape(B, SQ, D_MODEL)


# device time: 14371 ns/iter; 1.5542x vs baseline; 1.1661x over previous
import jax
import jax.numpy as jnp
from jax import lax
from jax.experimental import pallas as pl
from jax.experimental.pallas import tpu as pltpu

N_DEV = 4
B, SQ, SKV, HQ_LOCAL, DH = 2, 128, 128, 4, 64
D_MODEL = 512
D_HEADS = HQ_LOCAL * DH
M = B * SQ


def kernel(x, Wq, K_ext, V_ext, Wo):
    my = lax.axis_index("i")
    Wq_s = lax.dynamic_slice_in_dim(Wq, my * D_HEADS, D_HEADS, axis=1)
    Wo_s = lax.dynamic_slice_in_dim(Wo, my * D_HEADS, D_HEADS, axis=0)
    x2d = x.reshape(M, D_MODEL)

    def body(x_ref, wq_ref, k_ref, v_ref, wo_ref, out_ref,
             buf_a, buf_b, comm_ref, send_sems, recv_sems):
        my_pos = lax.axis_index("i")
        p1 = my_pos ^ 1
        p2 = 3 - my_pos

        barrier_sem = pltpu.get_barrier_semaphore()
        for nbr in [p1, p2]:
            pl.semaphore_signal(
                barrier_sem, inc=1,
                device_id=(nbr,), device_id_type=pl.DeviceIdType.MESH,
            )

        f32 = jnp.float32
        wqb = wq_ref[:].astype(jnp.bfloat16)
        wob = wo_ref[:].astype(jnp.bfloat16)

        rows = lax.broadcasted_iota(jnp.int32, (SQ, SKV), 0)
        cols = lax.broadcasted_iota(jnp.int32, (SQ, SKV), 1)
        mask = (cols // 64) <= (rows // 64)

        def half_partial(b):
            xb = x_ref[b * SQ:(b + 1) * SQ, :].astype(jnp.bfloat16)
            q_b = lax.dot(xb, wqb, preferred_element_type=f32)
            q_b = q_b.astype(jnp.bfloat16)
            heads = []
            for h in range(HQ_LOCAL):
                q = q_b[:, h * DH:(h + 1) * DH]
                k = k_ref[b, :, h, :].astype(jnp.bfloat16)
                v = v_ref[b, :, h, :].astype(jnp.bfloat16)
                s = lax.dot_general(
                    q, k, (((1,), (1,)), ((), ())),
                    preferred_element_type=f32,
                ) * 0.125
                w = jnp.exp(jnp.where(mask, s, -1e9))
                w = w / jnp.sum(w, axis=-1, keepdims=True)
                heads.append(
                    lax.dot(w.astype(jnp.bfloat16), v,
                            preferred_element_type=f32)
                )
            ctx_b = jnp.concatenate(heads, axis=1).astype(jnp.bfloat16)
            return lax.dot(ctx_b, wob, preferred_element_type=f32)

        def p1_rdma(half):
            return pltpu.make_async_remote_copy(
                src_ref=buf_a.at[half], dst_ref=comm_ref.at[half],
                send_sem=send_sems.at[half], recv_sem=recv_sems.at[half],
                device_id=(p1,), device_id_type=pl.DeviceIdType.MESH,
            )

        def p2_rdma(half):
            return pltpu.make_async_remote_copy(
                src_ref=buf_b.at[half], dst_ref=comm_ref.at[2 + half],
                send_sem=send_sems.at[2 + half],
                recv_sem=recv_sems.at[2 + half],
                device_id=(p2,), device_id_type=pl.DeviceIdType.MESH,
            )

        part0 = half_partial(0)
        buf_a[0, :, :] = part0.astype(jnp.bfloat16)
        pl.semaphore_wait(barrier_sem, 2)
        rdma1 = [p1_rdma(0), p1_rdma(1)]
        rdma2 = [p2_rdma(0), p2_rdma(1)]
        rdma1[0].start()

        part1 = half_partial(1)
        buf_a[1, :, :] = part1.astype(jnp.bfloat16)
        rdma1[1].start()

        rdma1[0].wait_recv()
        sum1_0 = part0 + comm_ref[0, :, :].astype(f32)
        buf_b[0, :, :] = sum1_0.astype(jnp.bfloat16)
        rdma2[0].start()

        rdma1[1].wait_recv()
        sum1_1 = part1 + comm_ref[1, :, :].astype(f32)
        buf_b[1, :, :] = sum1_1.astype(jnp.bfloat16)
        rdma2[1].start()

        rdma2[0].wait_recv()
        out_ref[0:SQ, :] = sum1_0 + comm_ref[2, :, :].astype(f32)
        rdma2[1].wait_recv()
        out_ref[SQ:M, :] = sum1_1 + comm_ref[3, :, :].astype(f32)

        for r in rdma1 + rdma2:
            r.wait_send()

    out2d = pl.pallas_call(
        body,
        out_shape=jax.ShapeDtypeStruct((M, D_MODEL), jnp.float32),
        in_specs=[pl.BlockSpec(memory_space=pltpu.VMEM)] * 5,
        out_specs=pl.BlockSpec(memory_space=pltpu.VMEM),
        scratch_shapes=[
            pltpu.VMEM((2, SQ, D_MODEL), jnp.bfloat16),
            pltpu.VMEM((2, SQ, D_MODEL), jnp.bfloat16),
            pltpu.VMEM((4, SQ, D_MODEL), jnp.bfloat16),
            pltpu.SemaphoreType.DMA((4,)),
            pltpu.SemaphoreType.DMA((4,)),
        ],
        compiler_params=pltpu.CompilerParams(collective_id=0),
    )(x2d, Wq_s, K_ext, V_ext, Wo_s)
    return out2d.reshape(B, SQ, D_MODEL)
